# Initial kernel scaffold; baseline (speedup 1.0000x reference)
import jax
import jax.numpy as jnp
from jax.experimental import pallas as pl


def kernel(boxes, scores):
    raise NotImplementedError("write your pallas kernel here")


# speedup vs baseline: 55.7448x; 55.7448x over previous
import jax
import jax.numpy as jnp
from jax import lax
from jax.experimental import pallas as pl
from jax.experimental.pallas import tpu as pltpu

_N = 5000
_T = 128
_NP = 5120
_M = _NP // _T
_THR = 0.6


def _nms_body(x1c_ref, y1c_ref, x2c_ref, y2c_ref, boxesT_ref, scores_ref,
              out_ref, keep_ref, s_ref):
    keep_ref[...] = jnp.ones((1, _NP), jnp.float32)

    rows_i = lax.broadcasted_iota(jnp.int32, (_T, _T), 0)
    cols_i = lax.broadcasted_iota(jnp.int32, (_T, _T), 1)
    upper = (cols_i > rows_i).astype(jnp.float32)
    eye = (rows_i == cols_i).astype(jnp.float32)
    lane = lax.broadcasted_iota(jnp.int32, (1, _T), 1)

    def block_step(i, carry):
        base = pl.multiple_of(i * _T, _T)
        x1c = x1c_ref[pl.ds(base, _T), :]
        y1c = y1c_ref[pl.ds(base, _T), :]
        x2c = x2c_ref[pl.ds(base, _T), :]
        y2c = y2c_ref[pl.ds(base, _T), :]
        areac = jnp.maximum(x2c - x1c, 0.0) * jnp.maximum(y2c - y1c, 0.0)

        def iou_vs_row(off):
            x1r = boxesT_ref[0:1, pl.ds(off, _T)]
            y1r = boxesT_ref[1:2, pl.ds(off, _T)]
            x2r = boxesT_ref[2:3, pl.ds(off, _T)]
            y2r = boxesT_ref[3:4, pl.ds(off, _T)]
            arear = jnp.maximum(x2r - x1r, 0.0) * jnp.maximum(y2r - y1r, 0.0)
            iw = jnp.maximum(jnp.minimum(x2c, x2r) - jnp.maximum(x1c, x1r), 0.0)
            ih = jnp.maximum(jnp.minimum(y2c, y2r) - jnp.maximum(y1c, y1r), 0.0)
            inter = iw * ih
            union = areac + arear - inter
            return inter / jnp.maximum(union, 1e-9)

        iou_ii = iou_vs_row(base)
        s_ref[...] = (iou_ii > _THR).astype(jnp.float32) * upper

        cur = keep_ref[0:1, pl.ds(base, _T)]
        for r in range(_T):
            onehot = (lane == r).astype(jnp.float32)
            a_r = jnp.sum(cur * onehot, axis=1, keepdims=True)
            row = s_ref[r:r + 1, :]
            cur = cur * (1.0 - row * a_r)
        keep_ref[0:1, pl.ds(base, _T)] = cur

        kb_col = jnp.sum(eye * cur, axis=1, keepdims=True)

        def cross_step(j, c):
            off = pl.multiple_of(j * _T, _T)
            iou_ij = iou_vs_row(off)
            sup = (iou_ij > _THR).astype(jnp.float32) * kb_col
            any_sup = jnp.max(sup, axis=0, keepdims=True)
            keep_ref[0:1, pl.ds(off, _T)] *= (1.0 - any_sup)
            return c

        lax.fori_loop(i + 1, _M, cross_step, 0)
        return carry

    lax.fori_loop(0, _M, block_step, 0)

    keep = keep_ref[0:1, :]
    out_ref[0:4, :] = boxesT_ref[...] * keep
    out_ref[4:5, :] = jnp.sqrt(jnp.maximum(scores_ref[...], 1e-8)) * keep


def kernel(boxes, scores):
    order = jnp.argsort(-scores)
    b = boxes[order].astype(jnp.float32)
    s = scores[order].astype(jnp.float32)
    pad = _NP - _N
    b = jnp.pad(b, ((0, pad), (0, 0)))
    s = jnp.pad(s, ((0, pad),))

    out5 = pl.pallas_call(
        _nms_body,
        out_shape=jax.ShapeDtypeStruct((5, _NP), jnp.float32),
        scratch_shapes=[
            pltpu.VMEM((1, _NP), jnp.float32),
            pltpu.VMEM((_T, _T), jnp.float32),
        ],
    )(b[:, 0:1], b[:, 1:2], b[:, 2:3], b[:, 3:4], b.T, s.reshape(1, _NP))
    return out5[:, :_N].T


# speedup vs baseline: 210.5081x; 3.7763x over previous
import jax
import jax.numpy as jnp
from jax import lax
from jax.experimental import pallas as pl
from jax.experimental.pallas import tpu as pltpu

_N = 5000
_T = 128
_NP = 5120
_M = _NP // _T
_THR = 0.6


def _nms_body(x1c_ref, y1c_ref, x2c_ref, y2c_ref, boxesT_ref, scores_ref,
              out_ref, keep_ref):
    keep_ref[...] = jnp.ones((1, _NP), jnp.float32)

    rows_i = lax.broadcasted_iota(jnp.int32, (_T, _T), 0)
    cols_i = lax.broadcasted_iota(jnp.int32, (_T, _T), 1)
    upper = (cols_i > rows_i).astype(jnp.float32)
    eye = (rows_i == cols_i).astype(jnp.float32)

    def to_col(v):
        return jnp.sum(eye * v, axis=1, keepdims=True)

    def block_step(i, carry):
        base = pl.multiple_of(i * _T, _T)
        x1c = x1c_ref[pl.ds(base, _T), :]
        y1c = y1c_ref[pl.ds(base, _T), :]
        x2c = x2c_ref[pl.ds(base, _T), :]
        y2c = y2c_ref[pl.ds(base, _T), :]
        areac = jnp.maximum(x2c - x1c, 0.0) * jnp.maximum(y2c - y1c, 0.0)

        def iou_vs_row(off):
            x1r = boxesT_ref[0:1, pl.ds(off, _T)]
            y1r = boxesT_ref[1:2, pl.ds(off, _T)]
            x2r = boxesT_ref[2:3, pl.ds(off, _T)]
            y2r = boxesT_ref[3:4, pl.ds(off, _T)]
            arear = jnp.maximum(x2r - x1r, 0.0) * jnp.maximum(y2r - y1r, 0.0)
            iw = jnp.maximum(jnp.minimum(x2c, x2r) - jnp.maximum(x1c, x1r), 0.0)
            ih = jnp.maximum(jnp.minimum(y2c, y2r) - jnp.maximum(y1c, y1r), 0.0)
            inter = iw * ih
            union = areac + arear - inter
            return inter / jnp.maximum(union, 1e-9)

        ki = keep_ref[0:1, pl.ds(base, _T)]
        iou_ii = iou_vs_row(base)
        s0 = (iou_ii > _THR).astype(jnp.float32) * upper * to_col(ki)

        def fix_cond(c):
            _s, changed = c
            return changed > 0.0

        def fix_body(c):
            s, _changed = c
            suppressed = jnp.max(s, axis=0, keepdims=True)
            can = ki * (1.0 - suppressed)
            new2 = jnp.max(s * to_col(can), axis=0, keepdims=True)
            s2 = s * (1.0 - to_col(new2))
            return (s2, jnp.sum(jnp.abs(s2 - s)))

        sf, _ = lax.while_loop(fix_cond, fix_body, (s0, jnp.float32(1.0)))
        supf = jnp.max(sf, axis=0, keepdims=True)
        canf = ki * (1.0 - supf)
        newf = jnp.max(sf * to_col(canf), axis=0, keepdims=True)
        cur = ki * (1.0 - newf)
        keep_ref[0:1, pl.ds(base, _T)] = cur
        kb_col = to_col(cur)

        def cross_step(j, c):
            off = pl.multiple_of(j * _T, _T)
            iou_ij = iou_vs_row(off)
            sup = (iou_ij > _THR).astype(jnp.float32) * kb_col
            any_sup = jnp.max(sup, axis=0, keepdims=True)
            keep_ref[0:1, pl.ds(off, _T)] *= (1.0 - any_sup)
            return c

        lax.fori_loop(i + 1, _M, cross_step, 0)
        return carry

    lax.fori_loop(0, _M, block_step, 0)

    keep = keep_ref[0:1, :]
    out_ref[0:4, :] = boxesT_ref[...] * keep
    out_ref[4:5, :] = jnp.sqrt(jnp.maximum(scores_ref[...], 1e-8)) * keep


def kernel(boxes, scores):
    order = jnp.argsort(-scores)
    b = boxes[order].astype(jnp.float32)
    s = scores[order].astype(jnp.float32)
    pad = _NP - _N
    b = jnp.pad(b, ((0, pad), (0, 0)))
    s = jnp.pad(s, ((0, pad),))

    out5 = pl.pallas_call(
        _nms_body,
        out_shape=jax.ShapeDtypeStruct((5, _NP), jnp.float32),
        scratch_shapes=[
            pltpu.VMEM((1, _NP), jnp.float32),
        ],
    )(b[:, 0:1], b[:, 1:2], b[:, 2:3], b[:, 3:4], b.T, s.reshape(1, _NP))
    return out5[:, :_N].T


# speedup vs baseline: 235.8552x; 1.1204x over previous
import jax
import jax.numpy as jnp
from jax import lax
from jax.experimental import pallas as pl
from jax.experimental.pallas import tpu as pltpu
from jax.experimental.pallas import tpu_sc as plsc

_N = 5000
_T = 128
_NP = 5120
_M = _NP // _T
_THR = 0.6

_NC, _NS = 2, 16
_NW = _NC * _NS
_BPW = _NP // _NW
_CH = _BPW // 2


def _sc_gather_body(table_hbm, idx_hbm, out_hbm, idx0, idx1, rows, sem):
    wid = lax.axis_index("s") * _NC + lax.axis_index("c")
    base = wid * _BPW
    pltpu.sync_copy(idx_hbm.at[pl.ds(base, _CH)], idx0)
    pltpu.sync_copy(idx_hbm.at[pl.ds(base + _CH, _CH)], idx1)
    c0 = pltpu.async_copy(table_hbm.at[idx0], rows.at[pl.ds(0, _CH)], sem)
    c1 = pltpu.async_copy(table_hbm.at[idx1], rows.at[pl.ds(_CH, _CH)], sem)
    c0.wait()
    c1.wait()
    pltpu.sync_copy(rows, out_hbm.at[pl.ds(base, _BPW)])


def _sc_gather(table, idx):
    return pl.kernel(
        _sc_gather_body,
        out_type=jax.ShapeDtypeStruct((_NP, 128), jnp.float32),
        mesh=plsc.VectorSubcoreMesh(core_axis_name="c", subcore_axis_name="s"),
        scratch_types=[
            pltpu.VMEM((_CH,), jnp.int32),
            pltpu.VMEM((_CH,), jnp.int32),
            pltpu.VMEM((_BPW, 128), jnp.float32),
            pltpu.SemaphoreType.DMA,
        ],
    )(table, idx)


def _nms_body(x1c_ref, y1c_ref, x2c_ref, y2c_ref, boxesT_ref, scores_ref,
              out_ref, keep_ref):
    keep_ref[...] = jnp.ones((1, _NP), jnp.float32)

    rows_i = lax.broadcasted_iota(jnp.int32, (_T, _T), 0)
    cols_i = lax.broadcasted_iota(jnp.int32, (_T, _T), 1)
    upper = (cols_i > rows_i).astype(jnp.float32)
    eye = (rows_i == cols_i).astype(jnp.float32)

    def to_col(v):
        return jnp.sum(eye * v, axis=1, keepdims=True)

    def block_step(i, carry):
        base = pl.multiple_of(i * _T, _T)
        x1c = x1c_ref[pl.ds(base, _T), :]
        y1c = y1c_ref[pl.ds(base, _T), :]
        x2c = x2c_ref[pl.ds(base, _T), :]
        y2c = y2c_ref[pl.ds(base, _T), :]
        areac = jnp.maximum(x2c - x1c, 0.0) * jnp.maximum(y2c - y1c, 0.0)

        def iou_vs_row(off):
            x1r = boxesT_ref[0:1, pl.ds(off, _T)]
            y1r = boxesT_ref[1:2, pl.ds(off, _T)]
            x2r = boxesT_ref[2:3, pl.ds(off, _T)]
            y2r = boxesT_ref[3:4, pl.ds(off, _T)]
            arear = jnp.maximum(x2r - x1r, 0.0) * jnp.maximum(y2r - y1r, 0.0)
            iw = jnp.maximum(jnp.minimum(x2c, x2r) - jnp.maximum(x1c, x1r), 0.0)
            ih = jnp.maximum(jnp.minimum(y2c, y2r) - jnp.maximum(y1c, y1r), 0.0)
            inter = iw * ih
            union = areac + arear - inter
            return inter / jnp.maximum(union, 1e-9)

        ki = keep_ref[0:1, pl.ds(base, _T)]
        iou_ii = iou_vs_row(base)
        s0 = (iou_ii > _THR).astype(jnp.float32) * upper * to_col(ki)

        def fix_cond(c):
            _s, changed = c
            return changed > 0.0

        def fix_body(c):
            s, _changed = c
            suppressed = jnp.max(s, axis=0, keepdims=True)
            can = ki * (1.0 - suppressed)
            new2 = jnp.max(s * to_col(can), axis=0, keepdims=True)
            s2 = s * (1.0 - to_col(new2))
            return (s2, jnp.sum(jnp.abs(s2 - s)))

        sf, _ = lax.while_loop(fix_cond, fix_body, (s0, jnp.float32(1.0)))
        supf = jnp.max(sf, axis=0, keepdims=True)
        canf = ki * (1.0 - supf)
        newf = jnp.max(sf * to_col(canf), axis=0, keepdims=True)
        cur = ki * (1.0 - newf)
        keep_ref[0:1, pl.ds(base, _T)] = cur
        kb_col = to_col(cur)

        def cross_step(j, c):
            off = pl.multiple_of(j * _T, _T)
            iou_ij = iou_vs_row(off)
            sup = (iou_ij > _THR).astype(jnp.float32) * kb_col
            any_sup = jnp.max(sup, axis=0, keepdims=True)
            keep_ref[0:1, pl.ds(off, _T)] *= (1.0 - any_sup)
            return c

        lax.fori_loop(i + 1, _M, cross_step, 0)
        return carry

    lax.fori_loop(0, _M, block_step, 0)

    keep = keep_ref[0:1, :]
    out_ref[0:4, :] = boxesT_ref[...] * keep
    out_ref[4:5, :] = jnp.sqrt(jnp.maximum(scores_ref[...], 1e-8)) * keep


def kernel(boxes, scores):
    order = jnp.argsort(-scores).astype(jnp.int32)
    pad = _NP - _N
    table = jnp.pad(
        jnp.concatenate([boxes.astype(jnp.float32),
                         scores.astype(jnp.float32)[:, None]], axis=1),
        ((0, pad), (0, 123)))
    idx = jnp.concatenate([order, jnp.arange(_N, _NP, dtype=jnp.int32)])
    sorted16 = _sc_gather(table, idx)
    b = sorted16[:, 0:4]
    s = sorted16[:, 4]

    out5 = pl.pallas_call(
        _nms_body,
        out_shape=jax.ShapeDtypeStruct((5, _NP), jnp.float32),
        scratch_shapes=[
            pltpu.VMEM((1, _NP), jnp.float32),
        ],
    )(b[:, 0:1], b[:, 1:2], b[:, 2:3], b[:, 3:4], b.T, s.reshape(1, _NP))
    return out5[:, :_N].T


# speedup vs baseline: 241.2808x; 1.0230x over previous
import jax
import jax.numpy as jnp
from jax import lax
from jax.experimental import pallas as pl
from jax.experimental.pallas import tpu as pltpu
from jax.experimental.pallas import tpu_sc as plsc

_N = 5000
_T = 128
_NP = 5120
_M = _NP // _T
_THR = 0.6
_W = 4
_NPW = _NP + (_W - 1) * _T

_NC, _NS = 2, 16
_NW = _NC * _NS
_BPW = _NP // _NW
_CH = _BPW // 2


def _sc_gather_body(table_hbm, idx_hbm, out_hbm, idx0, idx1, rows, sem):
    wid = lax.axis_index("s") * _NC + lax.axis_index("c")
    base = wid * _BPW
    pltpu.sync_copy(idx_hbm.at[pl.ds(base, _CH)], idx0)
    pltpu.sync_copy(idx_hbm.at[pl.ds(base + _CH, _CH)], idx1)
    c0 = pltpu.async_copy(table_hbm.at[idx0], rows.at[pl.ds(0, _CH)], sem)
    c1 = pltpu.async_copy(table_hbm.at[idx1], rows.at[pl.ds(_CH, _CH)], sem)
    c0.wait()
    c1.wait()
    pltpu.sync_copy(rows, out_hbm.at[pl.ds(base, _BPW)])


def _sc_gather(table, idx):
    return pl.kernel(
        _sc_gather_body,
        out_type=jax.ShapeDtypeStruct((_NP, 128), jnp.float32),
        mesh=plsc.VectorSubcoreMesh(core_axis_name="c", subcore_axis_name="s"),
        scratch_types=[
            pltpu.VMEM((_CH,), jnp.int32),
            pltpu.VMEM((_CH,), jnp.int32),
            pltpu.VMEM((_BPW, 128), jnp.float32),
            pltpu.SemaphoreType.DMA,
        ],
    )(table, idx)


def _nms_body(x1c_ref, y1c_ref, x2c_ref, y2c_ref, boxesT_ref, scores_ref,
              out_ref, keep_ref):
    keep_ref[...] = jnp.ones((1, _NPW), jnp.float32)

    rows_i = lax.broadcasted_iota(jnp.int32, (_T, _T), 0)
    cols_i = lax.broadcasted_iota(jnp.int32, (_T, _T), 1)
    upper = (cols_i > rows_i).astype(jnp.float32)
    eye = (rows_i == cols_i).astype(jnp.float32)

    def to_col(v):
        return jnp.sum(eye * v, axis=1, keepdims=True)

    def block_step(i, carry):
        base = pl.multiple_of(i * _T, _T)
        x1c = x1c_ref[pl.ds(base, _T), :]
        y1c = y1c_ref[pl.ds(base, _T), :]
        x2c = x2c_ref[pl.ds(base, _T), :]
        y2c = y2c_ref[pl.ds(base, _T), :]
        areac = jnp.maximum(x2c - x1c, 0.0) * jnp.maximum(y2c - y1c, 0.0)

        def iou_vs_row(off, width=_T):
            x1r = boxesT_ref[0:1, pl.ds(off, width)]
            y1r = boxesT_ref[1:2, pl.ds(off, width)]
            x2r = boxesT_ref[2:3, pl.ds(off, width)]
            y2r = boxesT_ref[3:4, pl.ds(off, width)]
            arear = jnp.maximum(x2r - x1r, 0.0) * jnp.maximum(y2r - y1r, 0.0)
            iw = jnp.maximum(jnp.minimum(x2c, x2r) - jnp.maximum(x1c, x1r), 0.0)
            ih = jnp.maximum(jnp.minimum(y2c, y2r) - jnp.maximum(y1c, y1r), 0.0)
            inter = iw * ih
            union = areac + arear - inter
            return inter / jnp.maximum(union, 1e-9)

        ki = keep_ref[0:1, pl.ds(base, _T)]
        iou_ii = iou_vs_row(base)
        s0 = (iou_ii > _THR).astype(jnp.float32) * upper * to_col(ki)

        def fix_cond(c):
            _s, changed = c
            return changed > 0.0

        def fix_body(c):
            s, _changed = c
            suppressed = jnp.max(s, axis=0, keepdims=True)
            can = ki * (1.0 - suppressed)
            new2 = jnp.max(s * to_col(can), axis=0, keepdims=True)
            s2 = s * (1.0 - to_col(new2))
            return (s2, jnp.sum(jnp.abs(s2 - s)))

        sf, _ = lax.while_loop(fix_cond, fix_body, (s0, jnp.float32(1.0)))
        supf = jnp.max(sf, axis=0, keepdims=True)
        canf = ki * (1.0 - supf)
        newf = jnp.max(sf * to_col(canf), axis=0, keepdims=True)
        cur = ki * (1.0 - newf)
        keep_ref[0:1, pl.ds(base, _T)] = cur
        kb_col = to_col(cur)

        def cross_step(w, c):
            off = pl.multiple_of((i + 1) * _T + w * (_W * _T), _T)
            iou_ij = iou_vs_row(off, _W * _T)
            sup = (iou_ij > _THR).astype(jnp.float32) * kb_col
            any_sup = jnp.max(sup, axis=0, keepdims=True)
            keep_ref[0:1, pl.ds(off, _W * _T)] *= (1.0 - any_sup)
            return c

        num_wide = (_M - 1 - i + _W - 1) // _W
        lax.fori_loop(0, num_wide, cross_step, 0)
        return carry

    lax.fori_loop(0, _M, block_step, 0)

    keep = keep_ref[0:1, 0:_NP]
    out_ref[0:4, :] = boxesT_ref[0:4, 0:_NP] * keep
    out_ref[4:5, :] = jnp.sqrt(jnp.maximum(scores_ref[...], 1e-8)) * keep


def kernel(boxes, scores):
    order = jnp.argsort(-scores).astype(jnp.int32)
    pad = _NP - _N
    table = jnp.pad(
        jnp.concatenate([boxes.astype(jnp.float32),
                         scores.astype(jnp.float32)[:, None]], axis=1),
        ((0, pad), (0, 123)))
    idx = jnp.concatenate([order, jnp.arange(_N, _NP, dtype=jnp.int32)])
    sorted16 = _sc_gather(table, idx)
    b = sorted16[:, 0:4]
    s = sorted16[:, 4]

    out5 = pl.pallas_call(
        _nms_body,
        out_shape=jax.ShapeDtypeStruct((5, _NP), jnp.float32),
        scratch_shapes=[
            pltpu.VMEM((1, _NPW), jnp.float32),
        ],
    )(b[:, 0:1], b[:, 1:2], b[:, 2:3], b[:, 3:4],
      jnp.pad(b.T, ((0, 0), (0, _NPW - _NP))), s.reshape(1, _NP))
    return out5[:, :_N].T


# speedup vs baseline: 249.6191x; 1.0346x over previous
import jax
import jax.numpy as jnp
from jax import lax
from jax.experimental import pallas as pl
from jax.experimental.pallas import tpu as pltpu
from jax.experimental.pallas import tpu_sc as plsc

_N = 5000
_T = 128
_NP = 5120
_M = _NP // _T
_THR = 0.6
_W = 4
_NPW = _NP + (_W - 1) * _T

_NC, _NS = 2, 16
_NW = _NC * _NS
_BPW = _NP // _NW
_CH = _BPW // 2


def _sc_gather_body(table_hbm, idx_hbm, out_hbm, idx0, idx1, rows, sem):
    wid = lax.axis_index("s") * _NC + lax.axis_index("c")
    base = wid * _BPW
    pltpu.sync_copy(idx_hbm.at[pl.ds(base, _CH)], idx0)
    pltpu.sync_copy(idx_hbm.at[pl.ds(base + _CH, _CH)], idx1)
    c0 = pltpu.async_copy(table_hbm.at[idx0], rows.at[pl.ds(0, _CH)], sem)
    c1 = pltpu.async_copy(table_hbm.at[idx1], rows.at[pl.ds(_CH, _CH)], sem)
    c0.wait()
    c1.wait()
    pltpu.sync_copy(rows, out_hbm.at[pl.ds(base, _BPW)])


def _sc_gather(table, idx):
    return pl.kernel(
        _sc_gather_body,
        out_type=jax.ShapeDtypeStruct((_NP, 128), jnp.float32),
        mesh=plsc.VectorSubcoreMesh(core_axis_name="c", subcore_axis_name="s"),
        scratch_types=[
            pltpu.VMEM((_CH,), jnp.int32),
            pltpu.VMEM((_CH,), jnp.int32),
            pltpu.VMEM((_BPW, 128), jnp.float32),
            pltpu.SemaphoreType.DMA,
        ],
    )(table, idx)


def _nms_body(x1c_ref, y1c_ref, x2c_ref, y2c_ref, boxesT_ref, scores_ref,
              out_ref, keep_ref, area_ref):
    keep_ref[...] = jnp.ones((1, _NPW), jnp.float32)
    area_ref[...] = (
        jnp.maximum(boxesT_ref[2:3, :] - boxesT_ref[0:1, :], 0.0)
        * jnp.maximum(boxesT_ref[3:4, :] - boxesT_ref[1:2, :], 0.0))

    rows_i = lax.broadcasted_iota(jnp.int32, (_T, _T), 0)
    cols_i = lax.broadcasted_iota(jnp.int32, (_T, _T), 1)
    upper = (cols_i > rows_i).astype(jnp.float32)
    eye = (rows_i == cols_i).astype(jnp.float32)

    def to_col(v):
        return jnp.sum(eye * v, axis=1, keepdims=True)

    def block_step(i, carry):
        base = pl.multiple_of(i * _T, _T)
        x1c = x1c_ref[pl.ds(base, _T), :]
        y1c = y1c_ref[pl.ds(base, _T), :]
        x2c = x2c_ref[pl.ds(base, _T), :]
        y2c = y2c_ref[pl.ds(base, _T), :]
        areac = jnp.maximum(x2c - x1c, 0.0) * jnp.maximum(y2c - y1c, 0.0)

        def iou_vs_row(off, width=_T):
            x1r = boxesT_ref[0:1, pl.ds(off, width)]
            y1r = boxesT_ref[1:2, pl.ds(off, width)]
            x2r = boxesT_ref[2:3, pl.ds(off, width)]
            y2r = boxesT_ref[3:4, pl.ds(off, width)]
            arear = area_ref[0:1, pl.ds(off, width)]
            iw = jnp.maximum(jnp.minimum(x2c, x2r) - jnp.maximum(x1c, x1r), 0.0)
            ih = jnp.maximum(jnp.minimum(y2c, y2r) - jnp.maximum(y1c, y1r), 0.0)
            inter = iw * ih
            union = areac + arear - inter
            return inter / jnp.maximum(union, 1e-9)

        ki = keep_ref[0:1, pl.ds(base, _T)]
        iou_ii = iou_vs_row(base)
        s0 = (iou_ii > _THR).astype(jnp.float32) * upper * to_col(ki)

        def fix_cond(c):
            _s, changed = c
            return changed > 0.0

        def fix_body(c):
            s, _changed = c
            suppressed = jnp.max(s, axis=0, keepdims=True)
            can = ki * (1.0 - suppressed)
            new2 = jnp.max(s * to_col(can), axis=0, keepdims=True)
            s2 = s * (1.0 - to_col(new2))
            return (s2, jnp.sum(jnp.abs(s2 - s)))

        sf, _ = lax.while_loop(fix_cond, fix_body, (s0, jnp.float32(1.0)))
        supf = jnp.max(sf, axis=0, keepdims=True)
        cur = ki * (1.0 - supf)
        keep_ref[0:1, pl.ds(base, _T)] = cur
        kb_col = to_col(cur)

        def cross_step(w, c):
            off = pl.multiple_of((i + 1) * _T + w * (_W * _T), _T)
            iou_ij = iou_vs_row(off, _W * _T)
            sup = (iou_ij > _THR).astype(jnp.float32) * kb_col
            any_sup = jnp.max(sup, axis=0, keepdims=True)
            keep_ref[0:1, pl.ds(off, _W * _T)] *= (1.0 - any_sup)
            return c

        num_wide = (_M - 1 - i + _W - 1) // _W
        lax.fori_loop(0, num_wide, cross_step, 0)
        return carry

    lax.fori_loop(0, _M, block_step, 0)

    keep = keep_ref[0:1, 0:_NP]
    out_ref[0:4, :] = boxesT_ref[0:4, 0:_NP] * keep
    out_ref[4:5, :] = jnp.sqrt(jnp.maximum(scores_ref[...], 1e-8)) * keep


def kernel(boxes, scores):
    order = jnp.argsort(-scores).astype(jnp.int32)
    pad = _NP - _N
    table = jnp.pad(
        jnp.concatenate([boxes.astype(jnp.float32),
                         scores.astype(jnp.float32)[:, None]], axis=1),
        ((0, pad), (0, 123)))
    idx = jnp.concatenate([order, jnp.arange(_N, _NP, dtype=jnp.int32)])
    sorted16 = _sc_gather(table, idx)
    b = sorted16[:, 0:4]
    s = sorted16[:, 4]

    out5 = pl.pallas_call(
        _nms_body,
        out_shape=jax.ShapeDtypeStruct((5, _NP), jnp.float32),
        scratch_shapes=[
            pltpu.VMEM((1, _NPW), jnp.float32),
            pltpu.VMEM((1, _NPW), jnp.float32),
        ],
    )(b[:, 0:1], b[:, 1:2], b[:, 2:3], b[:, 3:4],
      jnp.pad(b.T, ((0, 0), (0, _NPW - _NP))), s.reshape(1, _NP))
    return out5[:, :_N].T
